# baseline (speedup 1.0000x reference)
import jax
import jax.numpy as jnp
from jax.experimental import pallas as pl
from jax.experimental.pallas import tpu as pltpu

EPS_ = 1e-09
NGATES = 64
BT = 1024


def _gating_block(x_ref, w_ref, o_ref):
    logits = jnp.dot(x_ref[...], w_ref[...], preferred_element_type=jnp.float32)
    m = jnp.max(logits, axis=-1, keepdims=True)
    e = jnp.exp(logits - m)
    s = jnp.sum(e, axis=-1, keepdims=True)
    cols = jax.lax.broadcasted_iota(jnp.int32, e.shape, 1)
    i1 = jnp.min(jnp.where(e == 1.0, cols, NGATES), axis=-1, keepdims=True)
    mask1 = cols == i1
    e2 = jnp.where(mask1, 0.0, e)
    em2 = jnp.max(e2, axis=-1, keepdims=True)
    i2 = jnp.min(jnp.where(e2 == em2, cols, NGATES), axis=-1, keepdims=True)
    mask2 = cols == i2
    v1 = 1.0 / s
    v2 = em2 / s
    denom = v1 + v2 + EPS_
    out = jnp.where(mask1, v1 / denom, 0.0)
    out = jnp.where(mask2, v2 / denom, out)
    o_ref[...] = out


def kernel(x, w_gating):
    b, group, dim = x.shape
    n = b * group
    x2 = x.reshape(n, dim)
    grid = (n // BT,)
    out = pl.pallas_call(
        _gating_block,
        grid=grid,
        in_specs=[
            pl.BlockSpec((BT, dim), lambda i: (i, 0)),
            pl.BlockSpec((dim, NGATES), lambda i: (0, 0)),
        ],
        out_specs=pl.BlockSpec((BT, NGATES), lambda i: (i, 0)),
        out_shape=jax.ShapeDtypeStruct((n, NGATES), jnp.float32),
        compiler_params=pltpu.CompilerParams(
            dimension_semantics=("parallel",),
        ),
    )(x2, w_gating)
    return out.reshape(b, group, NGATES)


# speedup vs baseline: 1.0057x; 1.0057x over previous
import jax
import jax.numpy as jnp
from jax.experimental import pallas as pl
from jax.experimental.pallas import tpu as pltpu

EPS_ = 1e-09
NGATES = 64
BT = 1024


def _gating_block(x_ref, w_ref, o_ref):
    logits = jnp.dot(x_ref[...], w_ref[...], preferred_element_type=jnp.float32)
    m = jnp.max(logits, axis=-1, keepdims=True)
    e = jnp.exp(logits - m)
    s = jnp.sum(e, axis=-1, keepdims=True)
    cols = jax.lax.broadcasted_iota(jnp.int32, e.shape, 1)
    i1 = jnp.min(jnp.where(logits == m, cols, NGATES), axis=-1, keepdims=True)
    mask1 = cols == i1
    e2 = jnp.where(mask1, 0.0, e)
    em2 = jnp.max(e2, axis=-1, keepdims=True)
    i2 = jnp.min(jnp.where(e2 == em2, cols, NGATES), axis=-1, keepdims=True)
    mask2 = cols == i2
    v1 = 1.0 / s
    v2 = em2 / s
    denom = v1 + v2 + EPS_
    out = jnp.where(mask1, v1 / denom, 0.0)
    out = jnp.where(mask2, v2 / denom, out)
    o_ref[...] = out


def kernel(x, w_gating):
    b, group, dim = x.shape
    n = b * group
    x2 = x.reshape(n, dim)
    grid = (n // BT,)
    out = pl.pallas_call(
        _gating_block,
        grid=grid,
        in_specs=[
            pl.BlockSpec((BT, dim), lambda i: (i, 0)),
            pl.BlockSpec((dim, NGATES), lambda i: (0, 0)),
        ],
        out_specs=pl.BlockSpec((BT, NGATES), lambda i: (i, 0)),
        out_shape=jax.ShapeDtypeStruct((n, NGATES), jnp.float32),
        compiler_params=pltpu.CompilerParams(
            dimension_semantics=("parallel",),
        ),
    )(x2, w_gating)
    return out.reshape(b, group, NGATES)
